# baseline (speedup 1.0000x reference)
import jax
import jax.numpy as jnp
import numpy as np
from jax.experimental import pallas as pl
from jax.experimental.pallas import tpu as pltpu

_DEPTH = 10

_LVL = [np.array([0], dtype=np.int64)]
for _d in range(_DEPTH):
    _LVL.append(np.concatenate([2 * _LVL[_d] + 1, 2 * _LVL[_d] + 2]))

_XA_NODES = np.empty(1792, dtype=np.int64)
for _d in range(7):
    _XA_NODES[_d * 128:(_d + 1) * 128] = _LVL[_d][np.arange(128) % (2 ** _d)]
_XA_NODES[896:1024] = _LVL[7]
_XA_NODES[1024:1280] = _LVL[8]
_XA_NODES[1280:1792] = _LVL[9]
_XA_ROW_IDX = _XA_NODES.astype(np.int32)

_Z_NODES = np.full(2048, -1, dtype=np.int64)
for _d in range(7):
    _Z_NODES[2 ** _d - 1: 2 ** (_d + 1) - 1] = _LVL[_d]
_Z_NODES[128:256] = _LVL[7]
_Z_NODES[256:512] = _LVL[8]
_Z_NODES[512:1024] = _LVL[9]
_Z_NODES[1024:2048] = _LVL[10]
_W_COL_IDX = np.where(_Z_NODES >= 0, 513 + _Z_NODES, 512).astype(np.int32)

_PA = np.zeros((1792, 1023), np.float32)
_PA[np.arange(1792), _XA_ROW_IDX] = 1.0
_QW = np.zeros((2560, 2048), np.float32)
_QW[_W_COL_IDX, np.arange(2048)] = 1.0

_BT = 2048


def _rot(x, k):
    return pltpu.roll(x, (-k) % x.shape[1], 1)


def _clip01(x):
    return jnp.clip(x, 0.0, 1.0)


def _tree_kernel(x1_ref, x2_ref, a_ref, w_ref, pa_ref, qw_ref, out_ref,
                 ap_ref, wz_ref):
    f32 = jnp.float32
    bf16 = jnp.bfloat16
    bt = x1_ref.shape[0]

    @pl.when(pl.program_id(0) == 0)
    def _prep():
        a_bf = a_ref[...].astype(bf16)
        pa = pa_ref[...]
        ap_ref[...] = jax.lax.dot_general(
            pa, a_bf, (((1,), (0,)), ((), ())),
            preferred_element_type=f32).astype(bf16)
        wz_ref[...] = jax.lax.dot_general(
            w_ref[...].astype(bf16), qw_ref[...], (((1,), (0,)), ((), ())),
            preferred_element_type=f32).astype(bf16)

    x2e = jnp.concatenate(
        [x2_ref[...].astype(bf16), jnp.ones((bt, 1), bf16)], axis=1)
    xa = jax.lax.dot_general(
        x2e, ap_ref[...], (((1,), (1,)), ((), ())),
        preferred_element_type=f32)

    lane = jax.lax.broadcasted_iota(jnp.int32, (bt, 128), 1)
    bit = [(lane & (1 << s)) != 0 for s in range(7)]
    lvl = [(lane >= 2 ** d - 1) & (lane < 2 ** (d + 1) - 1) for d in range(7)]

    u = jnp.ones((bt, 128), f32)
    s_group = jnp.zeros((bt, 128), f32)
    for d in range(7):
        s_group = jnp.where(lvl[d], _rot(u, 1), s_group)
        xt = xa[:, d * 128:(d + 1) * 128]
        signed = jnp.where(bit[d], -xt, xt)
        u = jnp.minimum(u, _clip01(signed))
    s_group = jnp.where(lane == 127, 1.0, s_group)
    q7 = u

    xa7 = xa[:, 896:1024]
    xa8 = xa[:, 1024:1280]
    xa9 = xa[:, 1280:1792]
    q8 = jnp.concatenate(
        [jnp.minimum(q7, _clip01(xa7)), jnp.minimum(q7, _clip01(-xa7))], axis=1)
    q9 = jnp.concatenate(
        [jnp.minimum(q8, _clip01(xa8)), jnp.minimum(q8, _clip01(-xa8))], axis=1)
    q10 = jnp.concatenate(
        [jnp.minimum(q9, _clip01(xa9)), jnp.minimum(q9, _clip01(-xa9))], axis=1)
    bf = jnp.bfloat16
    z = jnp.concatenate(
        [s_group.astype(bf), q7.astype(bf), q8.astype(bf), q9.astype(bf),
         q10.astype(bf)], axis=1)

    out = jax.lax.dot_general(
        x1_ref[...].astype(bf), w_ref[:, 0:512].astype(bf),
        (((1,), (1,)), ((), ())), preferred_element_type=f32)
    out += jax.lax.dot_general(
        z, wz_ref[...], (((1,), (1,)), ((), ())),
        preferred_element_type=f32)
    out_ref[...] = out


@jax.jit
def kernel(X1, X2, A, W):
    batch, in1 = X1.shape
    out_dim = W.shape[0]
    f32 = jnp.float32

    grid = (batch // _BT,)
    out = pl.pallas_call(
        _tree_kernel,
        grid=grid,
        in_specs=[
            pl.BlockSpec((_BT, in1), lambda i: (i, 0)),
            pl.BlockSpec((_BT, X2.shape[1]), lambda i: (i, 0)),
            pl.BlockSpec(A.shape, lambda i: (0, 0)),
            pl.BlockSpec(W.shape, lambda i: (0, 0)),
            pl.BlockSpec(_PA.shape, lambda i: (0, 0)),
            pl.BlockSpec(_QW.shape, lambda i: (0, 0)),
        ],
        out_specs=pl.BlockSpec((_BT, out_dim), lambda i: (i, 0)),
        out_shape=jax.ShapeDtypeStruct((batch, out_dim), f32),
        scratch_shapes=[
            pltpu.VMEM((1792, 513), jnp.bfloat16),
            pltpu.VMEM((128, 2048), jnp.bfloat16),
        ],
        compiler_params=pltpu.CompilerParams(
            dimension_semantics=("arbitrary",)),
    )(X1.astype(f32), X2.astype(f32), A.astype(f32), W.astype(f32),
      jnp.asarray(_PA, jnp.bfloat16), jnp.asarray(_QW, jnp.bfloat16))
    return out


# speedup vs baseline: 1.0141x; 1.0141x over previous
import jax
import jax.numpy as jnp
import numpy as np
from jax.experimental import pallas as pl
from jax.experimental.pallas import tpu as pltpu

_DEPTH = 10

_LVL = [np.array([0], dtype=np.int64)]
for _d in range(_DEPTH):
    _LVL.append(np.concatenate([2 * _LVL[_d] + 1, 2 * _LVL[_d] + 2]))

_XA_NODES = np.empty(1792, dtype=np.int64)
for _d in range(7):
    _XA_NODES[_d * 128:(_d + 1) * 128] = _LVL[_d][np.arange(128) % (2 ** _d)]
_XA_NODES[896:1024] = _LVL[7]
_XA_NODES[1024:1280] = _LVL[8]
_XA_NODES[1280:1792] = _LVL[9]
_XA_ROW_IDX = _XA_NODES.astype(np.int32)

_Z_NODES = np.full(2048, -1, dtype=np.int64)
for _d in range(7):
    _Z_NODES[2 ** _d - 1: 2 ** (_d + 1) - 1] = _LVL[_d]
_Z_NODES[128:256] = _LVL[7]
_Z_NODES[256:512] = _LVL[8]
_Z_NODES[512:1024] = _LVL[9]
_Z_NODES[1024:2048] = _LVL[10]
_W_COL_IDX = np.where(_Z_NODES >= 0, 513 + _Z_NODES, 512).astype(np.int32)

_PA = np.zeros((1792, 1023), np.float32)
_PA[np.arange(1792), _XA_ROW_IDX] = 1.0
_QW = np.zeros((2560, 2048), np.float32)
_QW[_W_COL_IDX, np.arange(2048)] = 1.0

_BT = 2048


def _rot(x, k):
    return pltpu.roll(x, (-k) % x.shape[1], 1)


def _clip01(x):
    return jnp.clip(x, 0.0, 1.0)


def _tree_kernel(x1_ref, x2_ref, a_ref, w_ref, pa_ref, qw_ref, out_ref,
                 ap_ref, wz_ref):
    f32 = jnp.float32
    bf16 = jnp.bfloat16
    bt = x1_ref.shape[0]

    @pl.when(pl.program_id(0) == 0)
    def _prep():
        a_bf = a_ref[...].astype(bf16)
        pa = pa_ref[...]
        ap_ref[...] = jax.lax.dot_general(
            pa, a_bf, (((1,), (0,)), ((), ())),
            preferred_element_type=f32)
        wz_ref[...] = jax.lax.dot_general(
            w_ref[...].astype(bf16), qw_ref[...], (((1,), (0,)), ((), ())),
            preferred_element_type=f32)

    x2e = jnp.concatenate(
        [x2_ref[...], jnp.ones((bt, 1), f32)], axis=1)
    xa = jax.lax.dot_general(
        x2e, ap_ref[...], (((1,), (1,)), ((), ())),
        preferred_element_type=f32)

    lane = jax.lax.broadcasted_iota(jnp.int32, (bt, 128), 1)
    bit = [(lane & (1 << s)) != 0 for s in range(7)]
    lvl = [(lane >= 2 ** d - 1) & (lane < 2 ** (d + 1) - 1) for d in range(7)]

    u = jnp.ones((bt, 128), f32)
    s_group = jnp.zeros((bt, 128), f32)
    for d in range(7):
        s_group = jnp.where(lvl[d], _rot(u, 1), s_group)
        xt = xa[:, d * 128:(d + 1) * 128]
        signed = jnp.where(bit[d], -xt, xt)
        u = jnp.minimum(u, _clip01(signed))
    s_group = jnp.where(lane == 127, 1.0, s_group)
    q7 = u

    xa7 = xa[:, 896:1024]
    xa8 = xa[:, 1024:1280]
    xa9 = xa[:, 1280:1792]
    q8 = jnp.concatenate(
        [jnp.minimum(q7, _clip01(xa7)), jnp.minimum(q7, _clip01(-xa7))], axis=1)
    q9 = jnp.concatenate(
        [jnp.minimum(q8, _clip01(xa8)), jnp.minimum(q8, _clip01(-xa8))], axis=1)
    q10 = jnp.concatenate(
        [jnp.minimum(q9, _clip01(xa9)), jnp.minimum(q9, _clip01(-xa9))], axis=1)
    z = jnp.concatenate([s_group, q7, q8, q9, q10], axis=1)

    out = jax.lax.dot_general(
        x1_ref[...], w_ref[:, 0:512], (((1,), (1,)), ((), ())),
        preferred_element_type=f32)
    out += jax.lax.dot_general(
        z, wz_ref[...], (((1,), (1,)), ((), ())),
        preferred_element_type=f32)
    out_ref[...] = out


@jax.jit
def kernel(X1, X2, A, W):
    batch, in1 = X1.shape
    out_dim = W.shape[0]
    f32 = jnp.float32

    grid = (batch // _BT,)
    out = pl.pallas_call(
        _tree_kernel,
        grid=grid,
        in_specs=[
            pl.BlockSpec((_BT, in1), lambda i: (i, 0)),
            pl.BlockSpec((_BT, X2.shape[1]), lambda i: (i, 0)),
            pl.BlockSpec(A.shape, lambda i: (0, 0)),
            pl.BlockSpec(W.shape, lambda i: (0, 0)),
            pl.BlockSpec(_PA.shape, lambda i: (0, 0)),
            pl.BlockSpec(_QW.shape, lambda i: (0, 0)),
        ],
        out_specs=pl.BlockSpec((_BT, out_dim), lambda i: (i, 0)),
        out_shape=jax.ShapeDtypeStruct((batch, out_dim), f32),
        scratch_shapes=[
            pltpu.VMEM((1792, 513), f32),
            pltpu.VMEM((128, 2048), f32),
        ],
        compiler_params=pltpu.CompilerParams(
            dimension_semantics=("arbitrary",)),
    )(X1.astype(f32), X2.astype(f32), A.astype(f32), W.astype(f32),
      jnp.asarray(_PA, jnp.bfloat16), jnp.asarray(_QW, jnp.bfloat16))
    return out


# speedup vs baseline: 1.1236x; 1.1081x over previous
import jax
import jax.numpy as jnp
import numpy as np
from jax.experimental import pallas as pl
from jax.experimental.pallas import tpu as pltpu

_DEPTH = 10

_LVL = [np.array([0], dtype=np.int64)]
for _d in range(_DEPTH):
    _LVL.append(np.concatenate([2 * _LVL[_d] + 1, 2 * _LVL[_d] + 2]))

_XA_NODES = np.empty(1792, dtype=np.int64)
for _d in range(7):
    _XA_NODES[_d * 128:(_d + 1) * 128] = _LVL[_d][np.arange(128) % (2 ** _d)]
_XA_NODES[896:1024] = _LVL[7]
_XA_NODES[1024:1280] = _LVL[8]
_XA_NODES[1280:1792] = _LVL[9]
_XA_ROW_IDX = _XA_NODES.astype(np.int32)

_Z_NODES = np.full(2048, -1, dtype=np.int64)
for _d in range(7):
    _Z_NODES[2 ** _d - 1: 2 ** (_d + 1) - 1] = _LVL[_d]
_Z_NODES[128:256] = _LVL[7]
_Z_NODES[256:512] = _LVL[8]
_Z_NODES[512:1024] = _LVL[9]
_Z_NODES[1024:2048] = _LVL[10]
_W_COL_IDX = np.where(_Z_NODES >= 0, 513 + _Z_NODES, 512).astype(np.int32)

_PA = np.zeros((1792, 1023), np.float32)
_PA[np.arange(1792), _XA_ROW_IDX] = 1.0
_QW = np.zeros((2560, 2048), np.float32)
_QW[_W_COL_IDX, np.arange(2048)] = 1.0

_BT = 2048


def _rot(x, k):
    return pltpu.roll(x, (-k) % x.shape[1], 1)


def _clip01(x):
    return jnp.clip(x, 0.0, 1.0)


def _tree_kernel(x1_ref, x2_ref, a_ref, w_ref, pa_ref, qw_ref, out_ref,
                 ap_ref, ab_ref, wz_ref):
    f32 = jnp.float32
    bf16 = jnp.bfloat16
    bt = x1_ref.shape[0]

    @pl.when(pl.program_id(0) == 0)
    def _prep():
        a_bf = a_ref[...].astype(bf16)
        pa = pa_ref[...]
        ap_ref[...] = jax.lax.dot_general(
            pa, a_bf, (((1,), (0,)), ((), ())),
            preferred_element_type=f32)
        ab_ref[...] = jax.lax.dot_general(
            a_bf[:, 512:513], pa, (((0,), (1,)), ((), ())),
            preferred_element_type=f32)
        wz_ref[...] = jax.lax.dot_general(
            w_ref[...].astype(bf16), qw_ref[...], (((1,), (0,)), ((), ())),
            preferred_element_type=f32)

    xa = jax.lax.dot_general(
        x2_ref[...], ap_ref[:, 0:512], (((1,), (1,)), ((), ())),
        preferred_element_type=f32) + ab_ref[...]

    lane = jax.lax.broadcasted_iota(jnp.int32, (bt, 128), 1)
    bit = [(lane & (1 << s)) != 0 for s in range(7)]
    lvl = [(lane >= 2 ** d - 1) & (lane < 2 ** (d + 1) - 1) for d in range(7)]

    u = jnp.ones((bt, 128), f32)
    s_group = jnp.zeros((bt, 128), f32)
    for d in range(7):
        s_group = jnp.where(lvl[d], _rot(u, 1), s_group)
        xt = xa[:, d * 128:(d + 1) * 128]
        signed = jnp.where(bit[d], -xt, xt)
        u = jnp.minimum(u, _clip01(signed))
    s_group = jnp.where(lane == 127, 1.0, s_group)
    q7 = u

    xa7 = xa[:, 896:1024]
    xa8 = xa[:, 1024:1280]
    xa9 = xa[:, 1280:1792]
    q8 = jnp.concatenate(
        [jnp.minimum(q7, _clip01(xa7)), jnp.minimum(q7, _clip01(-xa7))], axis=1)
    q9 = jnp.concatenate(
        [jnp.minimum(q8, _clip01(xa8)), jnp.minimum(q8, _clip01(-xa8))], axis=1)
    q10 = jnp.concatenate(
        [jnp.minimum(q9, _clip01(xa9)), jnp.minimum(q9, _clip01(-xa9))], axis=1)
    z = jnp.concatenate([s_group, q7, q8, q9, q10], axis=1)

    out = jax.lax.dot_general(
        x1_ref[...], w_ref[:, 0:512], (((1,), (1,)), ((), ())),
        preferred_element_type=f32)
    out += jax.lax.dot_general(
        z, wz_ref[...], (((1,), (1,)), ((), ())),
        preferred_element_type=f32)
    out_ref[...] = out


@jax.jit
def kernel(X1, X2, A, W):
    batch, in1 = X1.shape
    out_dim = W.shape[0]
    f32 = jnp.float32

    grid = (batch // _BT,)
    out = pl.pallas_call(
        _tree_kernel,
        grid=grid,
        in_specs=[
            pl.BlockSpec((_BT, in1), lambda i: (i, 0)),
            pl.BlockSpec((_BT, X2.shape[1]), lambda i: (i, 0)),
            pl.BlockSpec(A.shape, lambda i: (0, 0)),
            pl.BlockSpec(W.shape, lambda i: (0, 0)),
            pl.BlockSpec(_PA.shape, lambda i: (0, 0)),
            pl.BlockSpec(_QW.shape, lambda i: (0, 0)),
        ],
        out_specs=pl.BlockSpec((_BT, out_dim), lambda i: (i, 0)),
        out_shape=jax.ShapeDtypeStruct((batch, out_dim), f32),
        scratch_shapes=[
            pltpu.VMEM((1792, 513), f32),
            pltpu.VMEM((1, 1792), f32),
            pltpu.VMEM((128, 2048), f32),
        ],
        compiler_params=pltpu.CompilerParams(
            dimension_semantics=("arbitrary",)),
    )(X1.astype(f32), X2.astype(f32), A.astype(f32), W.astype(f32),
      jnp.asarray(_PA, jnp.bfloat16), jnp.asarray(_QW, jnp.bfloat16))
    return out
